# baseline (speedup 1.0000x reference)
import jax
import jax.numpy as jnp
from jax import lax
from jax.experimental import pallas as pl
from jax.experimental.pallas import tpu as pltpu
from jax.experimental.pallas import tpu_sc as plsc

_N = 100000
_G = 512
_NC = 2
_NS = 16
_NW = _NC * _NS
_L = 16

_PER_W = 3128
_LAST_W = _N - (_NW - 1) * _PER_W
_VECS = (_PER_W + _L - 1) // _L
_PW_PAD = _VECS * _L
_FB = 2048


def _sc_body(pos_hbm, batch_hbm, part_hbm, forces_hbm,
             pos_v, b_v, e_v, i_v, z_v, f_v, acc_sh):
  c = lax.axis_index("c")
  s = lax.axis_index("s")
  wid = c * _NS + s
  base = wid * _PER_W
  count = jnp.where(wid == _NW - 1, _LAST_W, _PER_W)

  def _zfill(i, _):
    z_v[pl.ds(i * _L, _L)] = jnp.zeros((_L,), jnp.float32)
    return 0
  lax.fori_loop(0, _G // _L, _zfill, 0)

  @pl.when(s == 0)
  def _():
    pltpu.sync_copy(z_v, acc_sh)
  plsc.subcore_barrier()

  @pl.when(wid < _NW - 1)
  def _():
    pltpu.sync_copy(pos_hbm.at[pl.ds(base * 3, _PER_W * 3)], pos_v.at[pl.ds(0, _PER_W * 3)])
    pltpu.sync_copy(batch_hbm.at[pl.ds(base, _PER_W)], b_v.at[pl.ds(0, _PER_W)])

  @pl.when(wid == _NW - 1)
  def _():
    pltpu.sync_copy(pos_hbm.at[pl.ds(base * 3, _LAST_W * 3)], pos_v.at[pl.ds(0, _LAST_W * 3)])
    pltpu.sync_copy(batch_hbm.at[pl.ds(base, _LAST_W)], b_v.at[pl.ds(0, _LAST_W)])

  lanes = lax.iota(jnp.int32, _L)

  def _step(i, _):
    off = i * _L
    rows = off + lanes
    valid = rows < count
    rows_c = jnp.where(valid, rows, 0)
    y = plsc.load_gather(pos_v, [rows_c * 3 + 1])
    e = jnp.abs(y) * jnp.float32(0.1)
    e = jnp.where(valid, e, jnp.float32(0.0))
    b = b_v[pl.ds(off, _L)]
    b = jnp.where(valid, b, 0)
    e_v[pl.ds(off, _L)] = e
    i_v[pl.ds(off, _L)] = b
    return 0
  lax.fori_loop(0, _VECS, _step, 0)

  pltpu.sync_copy(e_v, acc_sh.at[i_v], add=True)

  def _ffill(i, _):
    f_v[pl.ds(i * _L, _L)] = jnp.full((_L,), 0.05, jnp.float32)
    return 0
  lax.fori_loop(0, _FB // _L, _ffill, 0)

  fbase = wid * (_PER_W * 3)
  for j in range(4):
    pltpu.sync_copy(f_v, forces_hbm.at[pl.ds(fbase + j * _FB, _FB)])

  @pl.when(wid < _NW - 1)
  def _():
    tail = _PER_W * 3 - 4 * _FB
    pltpu.sync_copy(f_v.at[pl.ds(0, tail)], forces_hbm.at[pl.ds(fbase + 4 * _FB, tail)])

  @pl.when(wid == _NW - 1)
  def _():
    tail = _LAST_W * 3 - 4 * _FB
    pltpu.sync_copy(f_v.at[pl.ds(0, tail)], forces_hbm.at[pl.ds(fbase + 4 * _FB, tail)])

  plsc.subcore_barrier()

  @pl.when(s == 0)
  def _():
    pltpu.sync_copy(acc_sh, z_v)
    pltpu.sync_copy(z_v, part_hbm.at[c])


def _combine_body(p_ref, o_ref):
  o_ref[...] = p_ref[0:1, :] + p_ref[1:2, :]


def kernel(positions, batch):
  pos_flat = positions.reshape(-1)

  mesh = plsc.VectorSubcoreMesh(
      core_axis_name="c", subcore_axis_name="s",
      num_cores=_NC, num_subcores=_NS)
  sc = pl.kernel(
      _sc_body,
      out_type=(
          jax.ShapeDtypeStruct((_NC, _G), jnp.float32),
          jax.ShapeDtypeStruct((_N * 3,), jnp.float32),
      ),
      mesh=mesh,
      compiler_params=pltpu.CompilerParams(needs_layout_passes=False),
      scratch_types=[
          pltpu.VMEM((_PER_W * 3,), jnp.float32),
          pltpu.VMEM((_PW_PAD,), jnp.int32),
          pltpu.VMEM((_PW_PAD,), jnp.float32),
          pltpu.VMEM((_PW_PAD,), jnp.int32),
          pltpu.VMEM((_G,), jnp.float32),
          pltpu.VMEM((_FB,), jnp.float32),
          pltpu.VMEM_SHARED((_G,), jnp.float32),
      ],
  )
  part, forces_flat = sc(pos_flat, batch)

  energies_row = pl.pallas_call(
      _combine_body,
      out_shape=jax.ShapeDtypeStruct((1, _G), jnp.float32),
  )(part)

  return (energies_row.reshape(_G, 1), forces_flat.reshape(_N, 3))
